# Initial kernel scaffold; baseline (speedup 1.0000x reference)
import jax
import jax.numpy as jnp
from jax.experimental import pallas as pl


def kernel(x, edge_index, batch_index, W1, b1, W2, b2, Wlin, blin):
    raise NotImplementedError("write your pallas kernel here")


# speedup vs baseline: 13.1553x; 13.1553x over previous
import functools

import jax
import jax.numpy as jnp
from jax import lax
from jax.experimental import pallas as pl
from jax.experimental.pallas import tpu as pltpu
from jax.experimental.pallas import tpu_sc as plsc

N = 10000
E = 320000
D = 128
ODIM = 138
G = 256

NC = 2
NS = 16
NW = NC * NS
EPW = E // NW
CHUNK = 80
NCHUNK = EPW // CHUNK
NB = 400
NBC = N // NB
NP = 10240
NB1 = NP // NS

RB = 1000
_F32 = jnp.float32

@functools.cache
def _build_sc_kernels():
    mesh = plsc.VectorSubcoreMesh(core_axis_name="c", subcore_axis_name="s")

    @functools.partial(
        pl.kernel,
        mesh=mesh,
        out_type=jax.ShapeDtypeStruct((NC * NP,), _F32),
        scratch_types=[
            pltpu.VMEM((CHUNK,), jnp.int32),
            pltpu.VMEM((CHUNK,), _F32),
            pltpu.VMEM((NB1,), _F32),
            pltpu.VMEM_SHARED((NP,), _F32),
        ],
    )
    def deg_sc(dst_hbm, ones_hbm, out_hbm, idx_v, ones_v, zb_v, acc_sh):
        cid = lax.axis_index("c")
        sid = lax.axis_index("s")
        w = cid * NS + sid
        pltpu.sync_copy(ones_hbm, ones_v)

        def zb(i, carry):
            zb_v[pl.ds(i * 16, 16)] = jnp.zeros((16,), _F32)
            return carry

        lax.fori_loop(0, NB1 // 16, zb, 0)
        pltpu.sync_copy(zb_v, acc_sh.at[pl.ds(sid * NB1, NB1)])
        plsc.subcore_barrier()

        def body(i, carry):
            base = w * EPW + i * CHUNK
            pltpu.sync_copy(dst_hbm.at[pl.ds(base, CHUNK)], idx_v)
            pltpu.sync_copy(ones_v, acc_sh.at[idx_v], add=True)
            return carry

        lax.fori_loop(0, NCHUNK, body, 0)
        plsc.subcore_barrier()
        pltpu.sync_copy(acc_sh.at[pl.ds(sid * NB1, NB1)],
                        out_hbm.at[pl.ds(cid * NP + sid * NB1, NB1)])

    @functools.partial(
        pl.kernel,
        mesh=mesh,
        out_type=jax.ShapeDtypeStruct((NC, N, D), _F32),
        scratch_types=[
            pltpu.VMEM((CHUNK,), jnp.int32),
            pltpu.VMEM((CHUNK,), jnp.int32),
            pltpu.VMEM((CHUNK, D), _F32),
            pltpu.VMEM_SHARED((N, D), _F32),
            pltpu.SemaphoreType.DMA,
        ],
    )
    def agg_sc(hs_hbm, src_hbm, dst_hbm, zeros_hbm, out_hbm,
               si_v, di_v, rows_v, acc_sh, sem):
        cid = lax.axis_index("c")
        sid = lax.axis_index("s")
        w = cid * NS + sid

        def init(k, carry):
            j = k * NS + sid
            @pl.when(j < NBC)
            def _():
                pltpu.sync_copy(zeros_hbm, acc_sh.at[pl.ds(j * NB, NB)])
            return carry

        lax.fori_loop(0, (NBC + NS - 1) // NS, init, 0)
        plsc.subcore_barrier()

        def body(i, carry):
            base = w * EPW + i * CHUNK
            pltpu.sync_copy(src_hbm.at[pl.ds(base, CHUNK)], si_v)
            pltpu.sync_copy(dst_hbm.at[pl.ds(base, CHUNK)], di_v)
            pltpu.async_copy(hs_hbm.at[si_v], rows_v, sem).wait()
            pltpu.sync_copy(rows_v, acc_sh.at[di_v], add=True)
            return carry

        lax.fori_loop(0, NCHUNK, body, 0)
        plsc.subcore_barrier()

        def drain(k, carry):
            j = k * NS + sid
            @pl.when(j < NBC)
            def _():
                pltpu.sync_copy(acc_sh.at[pl.ds(j * NB, NB)],
                                out_hbm.at[cid, pl.ds(j * NB, NB)])
            return carry

        lax.fori_loop(0, (NBC + NS - 1) // NS, drain, 0)

    return deg_sc, agg_sc


def _dinv_of(degp_ref):
    d = degp_ref[...]
    deg = d[:, 0] + d[:, 1] + 1.0
    return lax.rsqrt(deg)


def _tc1_body(x_ref, w1_ref, degp_ref, hs_ref):
    dinv = _dinv_of(degp_ref)
    h = jnp.dot(x_ref[...], w1_ref[...], preferred_element_type=_F32)
    hs_ref[...] = h * dinv[:, None]


def _tc2_body(aggp_ref, hs1_ref, degp_ref, b1_ref, w2_ref, hs2_ref):
    dinv = _dinv_of(degp_ref)
    tot = aggp_ref[0] + aggp_ref[1] + hs1_ref[...]
    h = jnp.maximum(tot * dinv[:, None] + b1_ref[...], 0.0)
    hs2_ref[...] = jnp.dot(h, w2_ref[...], preferred_element_type=_F32) * dinv[:, None]


def _tc3_body(aggp_ref, hs2_ref, degp_ref, b2_ref, bi_ref, wl_ref, bl_ref,
              out_ref, sums, cnts):
    i = pl.program_id(0)

    @pl.when(i == 0)
    def _init():
        sums[...] = jnp.zeros_like(sums)
        cnts[...] = jnp.zeros_like(cnts)

    dinv = _dinv_of(degp_ref)
    tot = aggp_ref[0] + aggp_ref[1] + hs2_ref[...]
    h = jnp.maximum(tot * dinv[:, None] + b2_ref[...], 0.0)
    ids = bi_ref[0, 0]
    gid = lax.broadcasted_iota(jnp.int32, (G, RB), 0)
    oh = (gid == ids[None, :]).astype(_F32)
    sums[...] += jnp.dot(oh, h, preferred_element_type=_F32)
    cnts[...] += jnp.dot(oh, jnp.ones((RB, D), _F32), preferred_element_type=_F32)

    @pl.when(i == pl.num_programs(0) - 1)
    def _fin():
        g = sums[...] / jnp.maximum(cnts[...], 1.0)
        out_ref[...] = jnp.dot(g, wl_ref[...], preferred_element_type=_F32) + bl_ref[...]


_degp_spec = pl.BlockSpec((RB, NC), lambda i: (i, 0))
_row_spec = pl.BlockSpec((RB, D), lambda i: (i, 0))
_aggp_spec = pl.BlockSpec((NC, RB, D), lambda i: (0, i, 0))
_w_spec = pl.BlockSpec((D, D), lambda i: (0, 0))
_b_spec = pl.BlockSpec((1, D), lambda i: (0, 0))

_tc1 = pl.pallas_call(
    _tc1_body,
    grid=(N // RB,),
    in_specs=[_row_spec, _w_spec, _degp_spec],
    out_specs=_row_spec,
    out_shape=jax.ShapeDtypeStruct((N, D), _F32),
)

_tc2 = pl.pallas_call(
    _tc2_body,
    grid=(N // RB,),
    in_specs=[_aggp_spec, _row_spec, _degp_spec, _b_spec, _w_spec],
    out_specs=_row_spec,
    out_shape=jax.ShapeDtypeStruct((N, D), _F32),
)

_tc3 = pl.pallas_call(
    _tc3_body,
    grid=(N // RB,),
    in_specs=[_aggp_spec, _row_spec, _degp_spec, _b_spec,
              pl.BlockSpec((1, 1, RB), lambda i: (i, 0, 0)),
              pl.BlockSpec((D, ODIM), lambda i: (0, 0)),
              pl.BlockSpec((1, ODIM), lambda i: (0, 0))],
    out_specs=pl.BlockSpec((G, ODIM), lambda i: (0, 0)),
    out_shape=jax.ShapeDtypeStruct((G, ODIM), _F32),
    scratch_shapes=[pltpu.VMEM((G, D), _F32), pltpu.VMEM((G, D), _F32)],
)


def kernel(x, edge_index, batch_index, W1, b1, W2, b2, Wlin, blin):
    src = edge_index[0].astype(jnp.int32)
    dst = edge_index[1].astype(jnp.int32)
    bi3d = batch_index.astype(jnp.int32).reshape(N // RB, 1, RB)
    ones_chunk = jnp.ones((CHUNK,), _F32)
    zeros_agg = jnp.zeros((NB, D), _F32)

    deg_sc, agg_sc = _build_sc_kernels()
    deg1d = deg_sc(dst, ones_chunk)
    degp = deg1d.reshape(NC, NP)[:, :N].transpose(1, 0)
    hs1 = _tc1(x, W1, degp)
    aggp1 = agg_sc(hs1, src, dst, zeros_agg)
    hs2 = _tc2(aggp1, hs1, degp, b1.reshape(1, D), W2)
    aggp2 = agg_sc(hs2, src, dst, zeros_agg)
    logits = _tc3(aggp2, hs2, degp, b2.reshape(1, D), bi3d,
                  Wlin, blin.reshape(1, ODIM))
    return logits


# speedup vs baseline: 28.8608x; 2.1939x over previous
import functools

import jax
import jax.numpy as jnp
from jax import lax
from jax.experimental import pallas as pl
from jax.experimental.pallas import tpu as pltpu
from jax.experimental.pallas import tpu_sc as plsc

N = 10000
E = 320000
D = 128
ODIM = 138
G = 256

NC = 2
NS = 16
NW = NC * NS
EPW = E // NW
PADE = 240
EPWP = EPW + PADE
CHUNK = 128
NCHUNK = EPWP // CHUNK
NBUF = 2
NBI = 4
NB = 400
NBC = N // NB
NP = 10240
NB1 = NP // NS

RB = 1000
_F32 = jnp.float32

@functools.cache
def _build_sc_kernels():
    mesh = plsc.VectorSubcoreMesh(core_axis_name="c", subcore_axis_name="s")

    @functools.partial(
        pl.kernel,
        mesh=mesh,
        out_type=jax.ShapeDtypeStruct((NC * NP,), _F32),
        scratch_types=[
            pltpu.VMEM((NCHUNK, CHUNK), jnp.int32),
            pltpu.VMEM((CHUNK,), _F32),
            pltpu.VMEM((NB1,), _F32),
            pltpu.VMEM_SHARED((NP,), _F32),
            pltpu.SemaphoreType.DMA,
        ],
    )
    def deg_sc(dst_hbm, ones_hbm, out_hbm, di_v, ones_v, zb_v, acc_sh, ssem):
        cid = lax.axis_index("c")
        sid = lax.axis_index("s")
        w = cid * NS + sid
        pltpu.sync_copy(ones_hbm, ones_v)
        pltpu.sync_copy(dst_hbm.at[w], di_v)

        def zb(i, carry):
            zb_v[pl.ds(i * 16, 16)] = jnp.zeros((16,), _F32)
            return carry

        lax.fori_loop(0, NB1 // 16, zb, 0)
        pltpu.sync_copy(zb_v, acc_sh.at[pl.ds(sid * NB1, NB1)])
        plsc.subcore_barrier()

        def body(i, carry):
            pltpu.async_copy(ones_v, acc_sh.at[di_v.at[i]], ssem, add=True)
            return carry

        lax.fori_loop(0, NCHUNK, body, 0)

        def drain(i, carry):
            pltpu.make_async_copy(ones_v, acc_sh.at[di_v.at[0]], ssem).wait()
            return carry

        lax.fori_loop(0, NCHUNK, drain, 0)
        plsc.subcore_barrier()
        pltpu.sync_copy(acc_sh.at[pl.ds(sid * NB1, NB1)],
                        out_hbm.at[pl.ds(cid * NP + sid * NB1, NB1)])

    @functools.partial(
        pl.kernel,
        mesh=mesh,
        out_type=jax.ShapeDtypeStruct((NC, N, D), _F32),
        scratch_types=(
            [pltpu.VMEM((CHUNK,), jnp.int32) for _ in range(2 * NBI)]
            + [
                pltpu.VMEM((NBUF, CHUNK, D), _F32),
                pltpu.VMEM_SHARED((NP, D), _F32),
                pltpu.SemaphoreType.DMA((NBUF,)),
                pltpu.SemaphoreType.DMA((NBUF,)),
                pltpu.SemaphoreType.DMA,
            ]
        ),
    )
    def agg_sc(hs_hbm, src_hbm, dst_hbm, zeros_hbm, out_hbm,
               si0, si1, si2, si3, di0, di1, di2, di3,
               rows, acc_sh, gsem, ssem, isem):
        si_l = [si0, si1, si2, si3]
        di_l = [di0, di1, di2, di3]
        cid = lax.axis_index("c")
        sid = lax.axis_index("s")
        w = cid * NS + sid

        def init(k, carry):
            j = k * NS + sid
            @pl.when(j < NBC)
            def _():
                pltpu.sync_copy(zeros_hbm.at[pl.ds(j * NB, NB)],
                                acc_sh.at[pl.ds(j * NB, NB)])
            return carry

        lax.fori_loop(0, (NBC + NS - 1) // NS, init, 0)
        plsc.subcore_barrier()

        def wait_gather(b):
            pltpu.make_async_copy(hs_hbm.at[pl.ds(0, CHUNK)], rows.at[b],
                                  gsem.at[b]).wait()

        def wait_scatter(b):
            pltpu.make_async_copy(hs_hbm.at[pl.ds(0, CHUNK)],
                                  acc_sh.at[pl.ds(0, CHUNK)], ssem.at[b]).wait()

        def wait_idx():
            pltpu.make_async_copy(src_hbm.at[w, 0], si0, isem).wait()
            pltpu.make_async_copy(src_hbm.at[w, 0], di0, isem).wait()

        pltpu.sync_copy(src_hbm.at[w, 0], si0)
        pltpu.sync_copy(dst_hbm.at[w, 0], di0)
        pltpu.sync_copy(src_hbm.at[w, 1], si1)
        pltpu.sync_copy(dst_hbm.at[w, 1], di1)
        pltpu.async_copy(hs_hbm.at[si0], rows.at[0], gsem.at[0])

        def round_(g, carry):
            for s4 in range(NBI):
                c = g * NBI + s4
                b = s4 % NBUF
                bn = (s4 + 1) % NBUF
                sn = (s4 + 1) % NBI
                sf = (s4 + 2) % NBI
                wait_gather(b)
                pltpu.async_copy(rows.at[b], acc_sh.at[di_l[s4]],
                                 ssem.at[b], add=True)
                cn = c + 1

                @pl.when(cn < NCHUNK)
                def _(cn=cn, bn=bn, sn=sn):
                    @pl.when(cn >= 2)
                    def __():
                        wait_idx()
                    @pl.when(cn >= NBUF)
                    def __():
                        wait_scatter(bn)
                    pltpu.async_copy(hs_hbm.at[si_l[sn]], rows.at[bn],
                                     gsem.at[bn])

                cf = c + 2

                @pl.when(cf < NCHUNK)
                def _(cf=cf, sf=sf):
                    pltpu.async_copy(src_hbm.at[w, cf], si_l[sf], isem)
                    pltpu.async_copy(dst_hbm.at[w, cf], di_l[sf], isem)
            return carry

        lax.fori_loop(0, NCHUNK // NBI, round_, 0)
        for b in range(NBUF):
            wait_scatter(b)
        plsc.subcore_barrier()

        def drain(k, carry):
            j = k * NS + sid
            @pl.when(j < NBC)
            def _():
                pltpu.sync_copy(acc_sh.at[pl.ds(j * NB, NB)],
                                out_hbm.at[cid, pl.ds(j * NB, NB)])
            return carry

        lax.fori_loop(0, (NBC + NS - 1) // NS, drain, 0)

    return deg_sc, agg_sc


def _dinv_of(degp_ref):
    d = degp_ref[...]
    deg = d[:, 0] + d[:, 1] + 1.0
    return lax.rsqrt(deg)


def _tc1_body(x_ref, w1_ref, degp_ref, hs_ref):
    dinv = _dinv_of(degp_ref)
    h = jnp.dot(x_ref[...], w1_ref[...], preferred_element_type=_F32)
    hs_ref[...] = h * dinv[:, None]


def _tc2_body(aggp_ref, hs1_ref, degp_ref, b1_ref, w2_ref, hs2_ref):
    dinv = _dinv_of(degp_ref)
    tot = aggp_ref[0] + aggp_ref[1] + hs1_ref[...]
    h = jnp.maximum(tot * dinv[:, None] + b1_ref[...], 0.0)
    hs2_ref[...] = jnp.dot(h, w2_ref[...], preferred_element_type=_F32) * dinv[:, None]


def _tc3_body(aggp_ref, hs2_ref, degp_ref, b2_ref, bi_ref, wl_ref, bl_ref,
              out_ref, sums, cnts):
    i = pl.program_id(0)

    @pl.when(i == 0)
    def _init():
        sums[...] = jnp.zeros_like(sums)
        cnts[...] = jnp.zeros_like(cnts)

    dinv = _dinv_of(degp_ref)
    tot = aggp_ref[0] + aggp_ref[1] + hs2_ref[...]
    h = jnp.maximum(tot * dinv[:, None] + b2_ref[...], 0.0)
    ids = bi_ref[0, 0]
    gid = lax.broadcasted_iota(jnp.int32, (G, RB), 0)
    oh = (gid == ids[None, :]).astype(_F32)
    sums[...] += jnp.dot(oh, h, preferred_element_type=_F32)
    cnts[...] += jnp.dot(oh, jnp.ones((RB, D), _F32), preferred_element_type=_F32)

    @pl.when(i == pl.num_programs(0) - 1)
    def _fin():
        g = sums[...] / jnp.maximum(cnts[...], 1.0)
        out_ref[...] = jnp.dot(g, wl_ref[...], preferred_element_type=_F32) + bl_ref[...]


_degp_spec = pl.BlockSpec((RB, NC), lambda i: (i, 0))
_row_spec = pl.BlockSpec((RB, D), lambda i: (i, 0))
_aggp_spec = pl.BlockSpec((NC, RB, D), lambda i: (0, i, 0))
_w_spec = pl.BlockSpec((D, D), lambda i: (0, 0))
_b_spec = pl.BlockSpec((1, D), lambda i: (0, 0))

_tc1 = pl.pallas_call(
    _tc1_body,
    grid=(N // RB,),
    in_specs=[_row_spec, _w_spec, _degp_spec],
    out_specs=_row_spec,
    out_shape=jax.ShapeDtypeStruct((N, D), _F32),
)

_tc2 = pl.pallas_call(
    _tc2_body,
    grid=(N // RB,),
    in_specs=[_aggp_spec, _row_spec, _degp_spec, _b_spec, _w_spec],
    out_specs=_row_spec,
    out_shape=jax.ShapeDtypeStruct((N, D), _F32),
)

_tc3 = pl.pallas_call(
    _tc3_body,
    grid=(N // RB,),
    in_specs=[_aggp_spec, _row_spec, _degp_spec, _b_spec,
              pl.BlockSpec((1, 1, RB), lambda i: (i, 0, 0)),
              pl.BlockSpec((D, ODIM), lambda i: (0, 0)),
              pl.BlockSpec((1, ODIM), lambda i: (0, 0))],
    out_specs=pl.BlockSpec((G, ODIM), lambda i: (0, 0)),
    out_shape=jax.ShapeDtypeStruct((G, ODIM), _F32),
    scratch_shapes=[pltpu.VMEM((G, D), _F32), pltpu.VMEM((G, D), _F32)],
)


def kernel(x, edge_index, batch_index, W1, b1, W2, b2, Wlin, blin):
    e0 = edge_index[0].astype(jnp.int32).reshape(NW, EPW)
    e1 = edge_index[1].astype(jnp.int32).reshape(NW, EPW)
    padk = jnp.arange(NW * PADE, dtype=jnp.int32).reshape(NW, PADE)
    src = jnp.concatenate([e0, padk % N], axis=1).reshape(NW, NCHUNK, CHUNK)
    dst = jnp.concatenate([e1, N + padk % PADE], axis=1).reshape(NW, NCHUNK, CHUNK)
    bi3d = batch_index.astype(jnp.int32).reshape(N // RB, 1, RB)
    ones_chunk = jnp.ones((CHUNK,), _F32)
    zeros_agg = jnp.zeros((N, D), _F32)

    deg_sc, agg_sc = _build_sc_kernels()
    deg1d = deg_sc(dst, ones_chunk)
    degp = deg1d.reshape(NC, NP)[:, :N].transpose(1, 0)
    hs1 = _tc1(x, W1, degp)
    aggp1 = agg_sc(hs1, src, dst, zeros_agg)
    hs2 = _tc2(aggp1, hs1, degp, b1.reshape(1, D), W2)
    aggp2 = agg_sc(hs2, src, dst, zeros_agg)
    logits = _tc3(aggp2, hs2, degp, b2.reshape(1, D), bi3d,
                  Wlin, blin.reshape(1, ODIM))
    return logits


# speedup vs baseline: 29.1157x; 1.0088x over previous
import functools

import jax
import jax.numpy as jnp
from jax import lax
from jax.experimental import pallas as pl
from jax.experimental.pallas import tpu as pltpu
from jax.experimental.pallas import tpu_sc as plsc

N = 10000
E = 320000
D = 128
ODIM = 138
G = 256

NC = 2
NS = 16
NW = NC * NS
EPW = E // NW
PADE = 240
EPWP = EPW + PADE
CHUNK = 128
NCHUNK = EPWP // CHUNK
NBUF = 2
NBI = 4
NB = 400
NBC = N // NB
NP = 10240
NB1 = NP // NS

RB = 1000
_F32 = jnp.float32

@functools.cache
def _build_sc_kernels():
    mesh = plsc.VectorSubcoreMesh(core_axis_name="c", subcore_axis_name="s")

    @functools.partial(
        pl.kernel,
        mesh=mesh,
        out_type=jax.ShapeDtypeStruct((NC * NP,), _F32),
        scratch_types=[
            pltpu.VMEM((NCHUNK, CHUNK), jnp.int32),
            pltpu.VMEM((CHUNK,), _F32),
            pltpu.VMEM((NB1,), _F32),
            pltpu.VMEM_SHARED((NP,), _F32),
            pltpu.SemaphoreType.DMA,
        ],
    )
    def deg_sc(dst_hbm, ones_hbm, out_hbm, di_v, ones_v, zb_v, acc_sh, ssem):
        cid = lax.axis_index("c")
        sid = lax.axis_index("s")
        w = cid * NS + sid
        pltpu.sync_copy(ones_hbm, ones_v)
        pltpu.sync_copy(dst_hbm.at[w], di_v)

        def zb(i, carry):
            zb_v[pl.ds(i * 16, 16)] = jnp.zeros((16,), _F32)
            return carry

        lax.fori_loop(0, NB1 // 16, zb, 0)
        pltpu.sync_copy(zb_v, acc_sh.at[pl.ds(sid * NB1, NB1)])
        plsc.subcore_barrier()

        def body(i, carry):
            pltpu.async_copy(ones_v, acc_sh.at[di_v.at[i]], ssem, add=True)
            return carry

        lax.fori_loop(0, NCHUNK, body, 0)

        def drain(i, carry):
            pltpu.make_async_copy(ones_v, acc_sh.at[di_v.at[0]], ssem).wait()
            return carry

        lax.fori_loop(0, NCHUNK, drain, 0)
        plsc.subcore_barrier()
        pltpu.sync_copy(acc_sh.at[pl.ds(sid * NB1, NB1)],
                        out_hbm.at[pl.ds(cid * NP + sid * NB1, NB1)])

    @functools.partial(
        pl.kernel,
        mesh=mesh,
        out_type=jax.ShapeDtypeStruct((NC, N, D), _F32),
        scratch_types=(
            [pltpu.VMEM((2, CHUNK), jnp.int32) for _ in range(NBI)]
            + [
                pltpu.VMEM((NBUF, CHUNK, D), _F32),
                pltpu.VMEM_SHARED((NP, D), _F32),
                pltpu.SemaphoreType.DMA((NBUF,)),
                pltpu.SemaphoreType.DMA((NBUF,)),
                pltpu.SemaphoreType.DMA,
            ]
        ),
    )
    def agg_sc(hs_hbm, sdi_hbm, zeros_hbm, out_hbm,
               sdi0, sdi1, sdi2, sdi3, rows, acc_sh, gsem, ssem, isem):
        sdi = [sdi0, sdi1, sdi2, sdi3]
        cid = lax.axis_index("c")
        sid = lax.axis_index("s")
        w = cid * NS + sid

        def init(k, carry):
            j = k * NS + sid
            @pl.when(j < NBC)
            def _():
                pltpu.sync_copy(zeros_hbm.at[pl.ds(j * NB, NB)],
                                acc_sh.at[pl.ds(j * NB, NB)])
            return carry

        lax.fori_loop(0, (NBC + NS - 1) // NS, init, 0)
        plsc.subcore_barrier()

        def wait_gather(b):
            pltpu.make_async_copy(hs_hbm.at[pl.ds(0, CHUNK)], rows.at[b],
                                  gsem.at[b]).wait()

        def wait_scatter(b):
            pltpu.make_async_copy(hs_hbm.at[pl.ds(0, CHUNK)],
                                  acc_sh.at[pl.ds(0, CHUNK)], ssem.at[b]).wait()

        def wait_idx():
            pltpu.make_async_copy(sdi_hbm.at[w, 0], sdi0, isem).wait()

        def scatter(c_static_mod, b):
            pltpu.async_copy(rows.at[b], acc_sh.at[sdi[c_static_mod].at[1]],
                             ssem.at[b], add=True)

        def gather(c_static_mod, b):
            pltpu.async_copy(hs_hbm.at[sdi[c_static_mod].at[0]], rows.at[b],
                             gsem.at[b])

        def load_idx(c, s):
            pltpu.async_copy(sdi_hbm.at[w, c], sdi[s], isem)

        pltpu.sync_copy(sdi_hbm.at[w, 0], sdi0)
        pltpu.sync_copy(sdi_hbm.at[w, 1], sdi1)
        gather(0, 0)
        wait_gather(0)
        scatter(0, 0)
        gather(1, 1)
        load_idx(2, 2)
        wait_gather(1)
        scatter(1, 1)
        wait_idx()
        wait_scatter(0)
        gather(2, 2 % NBUF)
        load_idx(3, 3)

        def steady(c, s4):
            b = s4 % NBUF
            bn = (s4 + 1) % NBUF
            sn = (s4 + 1) % NBI
            sf = (s4 + 2) % NBI
            wait_gather(b)
            scatter(s4, b)
            wait_idx()
            wait_scatter(bn)
            gather(sn, bn)
            load_idx(c + 2, sf)

        steady(2, 2)
        steady(3, 3)

        def round_(g, carry):
            for s4 in range(NBI):
                steady(g * NBI + s4, s4)
            return carry

        lax.fori_loop(1, NCHUNK // NBI - 1, round_, 0)
        cL = NCHUNK - 4
        steady(cL, cL % NBI)
        steady(cL + 1, (cL + 1) % NBI)
        s4 = (cL + 2) % NBI
        b = s4 % NBUF
        wait_gather(b)
        scatter(s4, b)
        wait_idx()
        wait_scatter((s4 + 1) % NBUF)
        gather((s4 + 1) % NBI, (s4 + 1) % NBUF)
        s4 = (cL + 3) % NBI
        b = s4 % NBUF
        wait_gather(b)
        scatter(s4, b)
        for b in range(NBUF):
            wait_scatter(b)
        plsc.subcore_barrier()

        def drain(k, carry):
            j = k * NS + sid
            @pl.when(j < NBC)
            def _():
                pltpu.sync_copy(acc_sh.at[pl.ds(j * NB, NB)],
                                out_hbm.at[cid, pl.ds(j * NB, NB)])
            return carry

        lax.fori_loop(0, (NBC + NS - 1) // NS, drain, 0)

    return deg_sc, agg_sc


def _dinv_of(degp_ref):
    d = degp_ref[...]
    deg = d[:, 0] + d[:, 1] + 1.0
    return lax.rsqrt(deg)


def _tc1_body(x_ref, w1_ref, degp_ref, hs_ref):
    dinv = _dinv_of(degp_ref)
    h = jnp.dot(x_ref[...], w1_ref[...], preferred_element_type=_F32)
    hs_ref[...] = h * dinv[:, None]


def _tc2_body(aggp_ref, hs1_ref, degp_ref, b1_ref, w2_ref, hs2_ref):
    dinv = _dinv_of(degp_ref)
    tot = aggp_ref[0] + aggp_ref[1] + hs1_ref[...]
    h = jnp.maximum(tot * dinv[:, None] + b1_ref[...], 0.0)
    hs2_ref[...] = jnp.dot(h, w2_ref[...], preferred_element_type=_F32) * dinv[:, None]


def _tc3_body(aggp_ref, hs2_ref, degp_ref, b2_ref, bi_ref, wl_ref, bl_ref,
              out_ref, sums, cnts):
    i = pl.program_id(0)

    @pl.when(i == 0)
    def _init():
        sums[...] = jnp.zeros_like(sums)
        cnts[...] = jnp.zeros_like(cnts)

    dinv = _dinv_of(degp_ref)
    tot = aggp_ref[0] + aggp_ref[1] + hs2_ref[...]
    h = jnp.maximum(tot * dinv[:, None] + b2_ref[...], 0.0)
    ids = bi_ref[0, 0]
    gid = lax.broadcasted_iota(jnp.int32, (G, RB), 0)
    oh = (gid == ids[None, :]).astype(_F32)
    sums[...] += jnp.dot(oh, h, preferred_element_type=_F32)
    cnts[...] += jnp.dot(oh, jnp.ones((RB, D), _F32), preferred_element_type=_F32)

    @pl.when(i == pl.num_programs(0) - 1)
    def _fin():
        g = sums[...] / jnp.maximum(cnts[...], 1.0)
        out_ref[...] = jnp.dot(g, wl_ref[...], preferred_element_type=_F32) + bl_ref[...]


_degp_spec = pl.BlockSpec((RB, NC), lambda i: (i, 0))
_row_spec = pl.BlockSpec((RB, D), lambda i: (i, 0))
_aggp_spec = pl.BlockSpec((NC, RB, D), lambda i: (0, i, 0))
_w_spec = pl.BlockSpec((D, D), lambda i: (0, 0))
_b_spec = pl.BlockSpec((1, D), lambda i: (0, 0))

_tc1 = pl.pallas_call(
    _tc1_body,
    grid=(N // RB,),
    in_specs=[_row_spec, _w_spec, _degp_spec],
    out_specs=_row_spec,
    out_shape=jax.ShapeDtypeStruct((N, D), _F32),
)

_tc2 = pl.pallas_call(
    _tc2_body,
    grid=(N // RB,),
    in_specs=[_aggp_spec, _row_spec, _degp_spec, _b_spec, _w_spec],
    out_specs=_row_spec,
    out_shape=jax.ShapeDtypeStruct((N, D), _F32),
)

_tc3 = pl.pallas_call(
    _tc3_body,
    grid=(N // RB,),
    in_specs=[_aggp_spec, _row_spec, _degp_spec, _b_spec,
              pl.BlockSpec((1, 1, RB), lambda i: (i, 0, 0)),
              pl.BlockSpec((D, ODIM), lambda i: (0, 0)),
              pl.BlockSpec((1, ODIM), lambda i: (0, 0))],
    out_specs=pl.BlockSpec((G, ODIM), lambda i: (0, 0)),
    out_shape=jax.ShapeDtypeStruct((G, ODIM), _F32),
    scratch_shapes=[pltpu.VMEM((G, D), _F32), pltpu.VMEM((G, D), _F32)],
)


def kernel(x, edge_index, batch_index, W1, b1, W2, b2, Wlin, blin):
    e0 = edge_index[0].astype(jnp.int32).reshape(NW, EPW)
    e1 = edge_index[1].astype(jnp.int32).reshape(NW, EPW)
    padk = jnp.arange(NW * PADE, dtype=jnp.int32).reshape(NW, PADE)
    src = jnp.concatenate([e0, padk % N], axis=1).reshape(NW, NCHUNK, CHUNK)
    dst = jnp.concatenate([e1, N + padk % PADE], axis=1).reshape(NW, NCHUNK, CHUNK)
    sdi = jnp.stack([src, dst], axis=2)
    bi3d = batch_index.astype(jnp.int32).reshape(N // RB, 1, RB)
    ones_chunk = jnp.ones((CHUNK,), _F32)
    zeros_agg = jnp.zeros((N, D), _F32)

    deg_sc, agg_sc = _build_sc_kernels()
    deg1d = deg_sc(dst, ones_chunk)
    degp = deg1d.reshape(NC, NP)[:, :N].transpose(1, 0)
    hs1 = _tc1(x, W1, degp)
    aggp1 = agg_sc(hs1, sdi, zeros_agg)
    hs2 = _tc2(aggp1, hs1, degp, b1.reshape(1, D), W2)
    aggp2 = agg_sc(hs2, sdi, zeros_agg)
    logits = _tc3(aggp2, hs2, degp, b2.reshape(1, D), bi3d,
                  Wlin, blin.reshape(1, ODIM))
    return logits


# speedup vs baseline: 29.4747x; 1.0123x over previous
import functools

import jax
import jax.numpy as jnp
from jax import lax
from jax.experimental import pallas as pl
from jax.experimental.pallas import tpu as pltpu
from jax.experimental.pallas import tpu_sc as plsc

N = 10000
E = 320000
D = 128
ODIM = 138
G = 256

NC = 2
NS = 16
NW = NC * NS
EPW = E // NW
PADE = 240
EPWP = EPW + PADE
CHUNK = 128
NCHUNK = EPWP // CHUNK
NBUF = 2
NBI = 4
NB = 400
NBC = N // NB
NP = 10240
NB1 = NP // NS

RB = 1000
_F32 = jnp.float32

@functools.cache
def _build_sc_kernels():
    mesh = plsc.VectorSubcoreMesh(core_axis_name="c", subcore_axis_name="s")

    @functools.partial(
        pl.kernel,
        mesh=mesh,
        out_type=jax.ShapeDtypeStruct((NC * NP,), _F32),
        scratch_types=[
            pltpu.VMEM((NCHUNK, CHUNK), jnp.int32),
            pltpu.VMEM((CHUNK,), _F32),
            pltpu.VMEM((NB1,), _F32),
            pltpu.VMEM_SHARED((NP,), _F32),
            pltpu.SemaphoreType.DMA,
        ],
    )
    def deg_sc(dst_hbm, ones_hbm, out_hbm, di_v, ones_v, zb_v, acc_sh, ssem):
        cid = lax.axis_index("c")
        sid = lax.axis_index("s")
        w = cid * NS + sid
        pltpu.sync_copy(ones_hbm, ones_v)
        pltpu.sync_copy(dst_hbm.at[w], di_v)

        def zb(i, carry):
            zb_v[pl.ds(i * 16, 16)] = jnp.zeros((16,), _F32)
            return carry

        lax.fori_loop(0, NB1 // 16, zb, 0)
        pltpu.sync_copy(zb_v, acc_sh.at[pl.ds(sid * NB1, NB1)])
        plsc.subcore_barrier()

        def body(i, carry):
            pltpu.async_copy(ones_v, acc_sh.at[di_v.at[i]], ssem, add=True)
            return carry

        lax.fori_loop(0, NCHUNK, body, 0)

        def drain(i, carry):
            pltpu.make_async_copy(ones_v, acc_sh.at[di_v.at[0]], ssem).wait()
            return carry

        lax.fori_loop(0, NCHUNK, drain, 0)
        plsc.subcore_barrier()
        pltpu.sync_copy(acc_sh.at[pl.ds(sid * NB1, NB1)],
                        out_hbm.at[pl.ds(cid * NP + sid * NB1, NB1)])

    @functools.partial(
        pl.kernel,
        mesh=mesh,
        out_type=jax.ShapeDtypeStruct((NC, N, D), _F32),
        scratch_types=(
            [pltpu.VMEM((2, CHUNK), jnp.int32) for _ in range(NBI)]
            + [
                pltpu.VMEM((NBUF, CHUNK, D), _F32),
                pltpu.VMEM_SHARED((NP, D), _F32),
                pltpu.SemaphoreType.DMA((NBUF,)),
                pltpu.SemaphoreType.DMA((NBUF,)),
                pltpu.SemaphoreType.DMA,
            ]
        ),
    )
    def agg_sc(hs_hbm, sdi_hbm, zeros_hbm, out_hbm,
               sdi0, sdi1, sdi2, sdi3, rows, acc_sh, gsem, ssem, isem):
        sdi = [sdi0, sdi1, sdi2, sdi3]
        cid = lax.axis_index("c")
        sid = lax.axis_index("s")
        w = cid * NS + sid

        def wait_gather(b):
            pltpu.make_async_copy(hs_hbm.at[pl.ds(0, CHUNK)], rows.at[b],
                                  gsem.at[b]).wait()

        def wait_scatter(b):
            pltpu.make_async_copy(hs_hbm.at[pl.ds(0, CHUNK)],
                                  acc_sh.at[pl.ds(0, CHUNK)], ssem.at[b]).wait()

        def wait_idx():
            pltpu.make_async_copy(sdi_hbm.at[w, 0], sdi0, isem).wait()

        def scatter(c_static_mod, b):
            pltpu.async_copy(rows.at[b], acc_sh.at[sdi[c_static_mod].at[1]],
                             ssem.at[b], add=True)

        def gather(c_static_mod, b):
            pltpu.async_copy(hs_hbm.at[sdi[c_static_mod].at[0]], rows.at[b],
                             gsem.at[b])

        def load_idx(c, s):
            pltpu.async_copy(sdi_hbm.at[w, c], sdi[s], isem)

        pltpu.sync_copy(sdi_hbm.at[w, 0], sdi0)
        pltpu.sync_copy(sdi_hbm.at[w, 1], sdi1)
        gather(0, 0)

        def init(k, carry):
            j = k * NS + sid
            @pl.when(j < NBC)
            def _():
                pltpu.sync_copy(zeros_hbm.at[pl.ds(j * NB, NB)],
                                acc_sh.at[pl.ds(j * NB, NB)])
            return carry

        lax.fori_loop(0, (NBC + NS - 1) // NS, init, 0)
        plsc.subcore_barrier()
        wait_gather(0)
        scatter(0, 0)
        gather(1, 1)
        load_idx(2, 2)
        wait_gather(1)
        scatter(1, 1)
        wait_idx()
        wait_scatter(0)
        gather(2, 2 % NBUF)
        load_idx(3, 3)

        def steady(c, s4):
            b = s4 % NBUF
            bn = (s4 + 1) % NBUF
            sn = (s4 + 1) % NBI
            sf = (s4 + 2) % NBI
            wait_gather(b)
            scatter(s4, b)
            wait_idx()
            wait_scatter(bn)
            gather(sn, bn)
            load_idx(c + 2, sf)

        steady(2, 2)
        steady(3, 3)

        def round_(g, carry):
            for s4 in range(NBI):
                steady(g * NBI + s4, s4)
            return carry

        lax.fori_loop(1, NCHUNK // NBI - 1, round_, 0)
        cL = NCHUNK - 4
        steady(cL, cL % NBI)
        steady(cL + 1, (cL + 1) % NBI)
        s4 = (cL + 2) % NBI
        b = s4 % NBUF
        wait_gather(b)
        scatter(s4, b)
        wait_idx()
        wait_scatter((s4 + 1) % NBUF)
        gather((s4 + 1) % NBI, (s4 + 1) % NBUF)
        s4 = (cL + 3) % NBI
        b = s4 % NBUF
        wait_gather(b)
        scatter(s4, b)
        for b in range(NBUF):
            wait_scatter(b)
        plsc.subcore_barrier()

        def drain(k, carry):
            j = k * NS + sid
            @pl.when(j < NBC)
            def _():
                pltpu.sync_copy(acc_sh.at[pl.ds(j * NB, NB)],
                                out_hbm.at[cid, pl.ds(j * NB, NB)])
            return carry

        lax.fori_loop(0, (NBC + NS - 1) // NS, drain, 0)

    return deg_sc, agg_sc


def _dinv_of(degp_ref):
    d = degp_ref[...]
    deg = d[:, 0] + d[:, 1] + 1.0
    return lax.rsqrt(deg)


def _tc1_body(x_ref, w1_ref, degp_ref, hs_ref):
    dinv = _dinv_of(degp_ref)
    h = jnp.dot(x_ref[...], w1_ref[...], preferred_element_type=_F32)
    hs_ref[...] = h * dinv[:, None]


def _tc2_body(aggp_ref, hs1_ref, degp_ref, b1_ref, w2_ref, hs2_ref):
    dinv = _dinv_of(degp_ref)
    tot = aggp_ref[0] + aggp_ref[1] + hs1_ref[...]
    h = jnp.maximum(tot * dinv[:, None] + b1_ref[...], 0.0)
    hs2_ref[...] = jnp.dot(h, w2_ref[...], preferred_element_type=_F32) * dinv[:, None]


def _tc3_body(aggp_ref, hs2_ref, degp_ref, b2_ref, bi_ref, wl_ref, bl_ref,
              out_ref, sums, cnts):
    i = pl.program_id(0)

    @pl.when(i == 0)
    def _init():
        sums[...] = jnp.zeros_like(sums)
        cnts[...] = jnp.zeros_like(cnts)

    dinv = _dinv_of(degp_ref)
    tot = aggp_ref[0] + aggp_ref[1] + hs2_ref[...]
    h = jnp.maximum(tot * dinv[:, None] + b2_ref[...], 0.0)
    ids = bi_ref[0, 0]
    gid = lax.broadcasted_iota(jnp.int32, (G, RB), 0)
    oh = (gid == ids[None, :]).astype(_F32)
    sums[...] += jnp.dot(oh, h, preferred_element_type=_F32)
    cnts[...] += jnp.broadcast_to(jnp.sum(oh, axis=1)[:, None], (G, D))

    @pl.when(i == pl.num_programs(0) - 1)
    def _fin():
        g = sums[...] / jnp.maximum(cnts[...], 1.0)
        out_ref[...] = jnp.dot(g, wl_ref[...], preferred_element_type=_F32) + bl_ref[...]


_degp_spec = pl.BlockSpec((RB, NC), lambda i: (i, 0))
_row_spec = pl.BlockSpec((RB, D), lambda i: (i, 0))
_aggp_spec = pl.BlockSpec((NC, RB, D), lambda i: (0, i, 0))
_w_spec = pl.BlockSpec((D, D), lambda i: (0, 0))
_b_spec = pl.BlockSpec((1, D), lambda i: (0, 0))

_tc1 = pl.pallas_call(
    _tc1_body,
    grid=(N // RB,),
    in_specs=[_row_spec, _w_spec, _degp_spec],
    out_specs=_row_spec,
    out_shape=jax.ShapeDtypeStruct((N, D), _F32),
)

_tc2 = pl.pallas_call(
    _tc2_body,
    grid=(N // RB,),
    in_specs=[_aggp_spec, _row_spec, _degp_spec, _b_spec, _w_spec],
    out_specs=_row_spec,
    out_shape=jax.ShapeDtypeStruct((N, D), _F32),
)

_tc3 = pl.pallas_call(
    _tc3_body,
    grid=(N // RB,),
    in_specs=[_aggp_spec, _row_spec, _degp_spec, _b_spec,
              pl.BlockSpec((1, 1, RB), lambda i: (i, 0, 0)),
              pl.BlockSpec((D, ODIM), lambda i: (0, 0)),
              pl.BlockSpec((1, ODIM), lambda i: (0, 0))],
    out_specs=pl.BlockSpec((G, ODIM), lambda i: (0, 0)),
    out_shape=jax.ShapeDtypeStruct((G, ODIM), _F32),
    scratch_shapes=[pltpu.VMEM((G, D), _F32), pltpu.VMEM((G, D), _F32)],
)


def kernel(x, edge_index, batch_index, W1, b1, W2, b2, Wlin, blin):
    e0 = edge_index[0].astype(jnp.int32).reshape(NW, EPW)
    e1 = edge_index[1].astype(jnp.int32).reshape(NW, EPW)
    padk = jnp.arange(NW * PADE, dtype=jnp.int32).reshape(NW, PADE)
    src = jnp.concatenate([e0, padk % N], axis=1).reshape(NW, NCHUNK, CHUNK)
    dst = jnp.concatenate([e1, N + padk % PADE], axis=1).reshape(NW, NCHUNK, CHUNK)
    sdi = jnp.stack([src, dst], axis=2)
    bi3d = batch_index.astype(jnp.int32).reshape(N // RB, 1, RB)
    ones_chunk = jnp.ones((CHUNK,), _F32)
    zeros_agg = jnp.zeros((N, D), _F32)

    deg_sc, agg_sc = _build_sc_kernels()
    deg1d = deg_sc(dst, ones_chunk)
    degp = deg1d.reshape(NC, NP)[:, :N].transpose(1, 0)
    hs1 = _tc1(x, W1, degp)
    aggp1 = agg_sc(hs1, sdi, zeros_agg)
    hs2 = _tc2(aggp1, hs1, degp, b1.reshape(1, D), W2)
    aggp2 = agg_sc(hs2, sdi, zeros_agg)
    logits = _tc3(aggp2, hs2, degp, b2.reshape(1, D), bi3d,
                  Wlin, blin.reshape(1, ODIM))
    return logits


# speedup vs baseline: 35.6725x; 1.2103x over previous
import functools

import jax
import jax.numpy as jnp
from jax import lax
from jax.experimental import pallas as pl
from jax.experimental.pallas import tpu as pltpu
from jax.experimental.pallas import tpu_sc as plsc

N = 10000
E = 320000
D = 128
ODIM = 138
G = 256

NC = 2
NS = 16
NW = NC * NS
EPW = E // NW
PADE = 240
EPWP = EPW + PADE
CHUNK = 128
NCHUNK = EPWP // CHUNK
NBUF = 2
NBI = 4
NB = 400
NBC = N // NB
NP = 10240
NB1 = NP // NS

RB = 1000
_F32 = jnp.float32

@functools.cache
def _build_sc_kernels():
    mesh = plsc.VectorSubcoreMesh(core_axis_name="c", subcore_axis_name="s")

    @functools.partial(
        pl.kernel,
        mesh=mesh,
        out_type=jax.ShapeDtypeStruct((NC * NP,), _F32),
        scratch_types=[
            pltpu.VMEM((NCHUNK, CHUNK), jnp.int32),
            pltpu.VMEM((CHUNK,), _F32),
            pltpu.VMEM((NB1,), _F32),
            pltpu.VMEM_SHARED((NP,), _F32),
            pltpu.SemaphoreType.DMA,
        ],
    )
    def deg_sc(dst_hbm, ones_hbm, out_hbm, di_v, ones_v, zb_v, acc_sh, ssem):
        cid = lax.axis_index("c")
        sid = lax.axis_index("s")
        w = cid * NS + sid
        pltpu.sync_copy(ones_hbm, ones_v)
        pltpu.sync_copy(dst_hbm.at[w], di_v)

        def zb(i, carry):
            zb_v[pl.ds(i * 16, 16)] = jnp.zeros((16,), _F32)
            return carry

        lax.fori_loop(0, NB1 // 16, zb, 0)
        pltpu.sync_copy(zb_v, acc_sh.at[pl.ds(sid * NB1, NB1)])
        plsc.subcore_barrier()

        def body(i, carry):
            pltpu.async_copy(ones_v, acc_sh.at[di_v.at[i]], ssem, add=True)
            return carry

        lax.fori_loop(0, NCHUNK, body, 0)

        def drain(i, carry):
            pltpu.make_async_copy(ones_v, acc_sh.at[di_v.at[0]], ssem).wait()
            return carry

        lax.fori_loop(0, NCHUNK, drain, 0)
        plsc.subcore_barrier()
        pltpu.sync_copy(acc_sh.at[pl.ds(sid * NB1, NB1)],
                        out_hbm.at[pl.ds(cid * NP + sid * NB1, NB1)])

    @functools.partial(
        pl.kernel,
        mesh=mesh,
        out_type=jax.ShapeDtypeStruct((NC, N, D), _F32),
        scratch_types=(
            [pltpu.VMEM((2, CHUNK), jnp.int32) for _ in range(NBI)]
            + [
                pltpu.VMEM((NBUF, CHUNK, D), _F32),
                pltpu.VMEM_SHARED((NP, D), _F32),
                pltpu.SemaphoreType.DMA((NBUF,)),
                pltpu.SemaphoreType.DMA((NBUF,)),
                pltpu.SemaphoreType.DMA,
            ]
        ),
    )
    def agg_sc(hs_hbm, sdi_hbm, zeros_hbm, out_hbm,
               sdi0, sdi1, sdi2, sdi3, rows, acc_sh, gsem, ssem, isem):
        sdi = [sdi0, sdi1, sdi2, sdi3]
        cid = lax.axis_index("c")
        sid = lax.axis_index("s")
        w = cid * NS + sid

        def wait_gather(b):
            pltpu.make_async_copy(hs_hbm.at[pl.ds(0, CHUNK)], rows.at[b],
                                  gsem.at[b]).wait()

        def wait_scatter(b):
            pltpu.make_async_copy(hs_hbm.at[pl.ds(0, CHUNK)],
                                  acc_sh.at[pl.ds(0, CHUNK)], ssem.at[b]).wait()

        def wait_idx():
            pltpu.make_async_copy(sdi_hbm.at[w, 0], sdi0, isem).wait()

        def scatter(c_static_mod, b):
            pltpu.async_copy(rows.at[b], acc_sh.at[sdi[c_static_mod].at[1]],
                             ssem.at[b], add=True)

        def gather(c_static_mod, b):
            pltpu.async_copy(hs_hbm.at[sdi[c_static_mod].at[0]], rows.at[b],
                             gsem.at[b])

        def load_idx(c, s):
            pltpu.async_copy(sdi_hbm.at[w, c], sdi[s], isem)

        pltpu.sync_copy(sdi_hbm.at[w, 0], sdi0)
        pltpu.sync_copy(sdi_hbm.at[w, 1], sdi1)
        gather(0, 0)

        def init(k, carry):
            j = k * NS + sid
            @pl.when(j < NBC)
            def _():
                pltpu.sync_copy(zeros_hbm.at[pl.ds(j * NB, NB)],
                                acc_sh.at[pl.ds(j * NB, NB)])
            return carry

        lax.fori_loop(0, (NBC + NS - 1) // NS, init, 0)
        plsc.subcore_barrier()
        gather(1, 1)
        load_idx(2, 2)
        wait_gather(0)
        scatter(0, 0)
        wait_idx()
        wait_scatter(0)
        gather(2, 2 % NBUF)
        load_idx(3, 3)
        wait_gather(1)
        scatter(1, 1)

        def steady(c, s4):
            b = s4 % NBUF
            bn = (s4 + 1) % NBUF
            sn = (s4 + 1) % NBI
            sf = (s4 + 2) % NBI
            wait_idx()
            wait_scatter(bn)
            gather(sn, bn)
            load_idx(c + 2, sf)
            wait_gather(b)
            scatter(s4, b)

        steady(2, 2)
        steady(3, 3)

        def round_(g, carry):
            for s4 in range(NBI):
                steady(g * NBI + s4, s4)
            return carry

        lax.fori_loop(1, NCHUNK // NBI - 1, round_, 0)
        cL = NCHUNK - 4
        steady(cL, cL % NBI)
        steady(cL + 1, (cL + 1) % NBI)
        s4 = (cL + 2) % NBI
        b = s4 % NBUF
        wait_idx()
        wait_scatter((s4 + 1) % NBUF)
        gather((s4 + 1) % NBI, (s4 + 1) % NBUF)
        wait_gather(b)
        scatter(s4, b)
        s4 = (cL + 3) % NBI
        b = s4 % NBUF
        wait_gather(b)
        scatter(s4, b)
        for b in range(NBUF):
            wait_scatter(b)
        plsc.subcore_barrier()

        def drain(k, carry):
            j = k * NS + sid
            @pl.when(j < NBC)
            def _():
                pltpu.sync_copy(acc_sh.at[pl.ds(j * NB, NB)],
                                out_hbm.at[cid, pl.ds(j * NB, NB)])
            return carry

        lax.fori_loop(0, (NBC + NS - 1) // NS, drain, 0)

    return deg_sc, agg_sc


def _dinv_of(degp_ref):
    d = degp_ref[...]
    deg = d[:, 0] + d[:, 1] + 1.0
    return lax.rsqrt(deg)


def _tc1_body(x_ref, w1_ref, degp_ref, hs_ref):
    dinv = _dinv_of(degp_ref)
    h = jnp.dot(x_ref[...], w1_ref[...], preferred_element_type=_F32)
    hs_ref[...] = h * dinv[:, None]


def _tc2_body(aggp_ref, hs1_ref, degp_ref, b1_ref, w2_ref, hs2_ref):
    dinv = _dinv_of(degp_ref)
    tot = aggp_ref[0] + aggp_ref[1] + hs1_ref[...]
    h = jnp.maximum(tot * dinv[:, None] + b1_ref[...], 0.0)
    hs2_ref[...] = jnp.dot(h, w2_ref[...], preferred_element_type=_F32) * dinv[:, None]


def _tc3_body(aggp_ref, hs2_ref, degp_ref, b2_ref, bi_ref, wl_ref, bl_ref,
              out_ref, sums, cnts):
    i = pl.program_id(0)

    @pl.when(i == 0)
    def _init():
        sums[...] = jnp.zeros_like(sums)
        cnts[...] = jnp.zeros_like(cnts)

    dinv = _dinv_of(degp_ref)
    tot = aggp_ref[0] + aggp_ref[1] + hs2_ref[...]
    h = jnp.maximum(tot * dinv[:, None] + b2_ref[...], 0.0)
    ids = bi_ref[0, 0]
    gid = lax.broadcasted_iota(jnp.int32, (G, RB), 0)
    oh = (gid == ids[None, :]).astype(_F32)
    sums[...] += jnp.dot(oh, h, preferred_element_type=_F32)
    cnts[...] += jnp.broadcast_to(jnp.sum(oh, axis=1)[:, None], (G, D))

    @pl.when(i == pl.num_programs(0) - 1)
    def _fin():
        g = sums[...] / jnp.maximum(cnts[...], 1.0)
        out_ref[...] = jnp.dot(g, wl_ref[...], preferred_element_type=_F32) + bl_ref[...]


_degp_spec = pl.BlockSpec((RB, NC), lambda i: (i, 0))
_row_spec = pl.BlockSpec((RB, D), lambda i: (i, 0))
_aggp_spec = pl.BlockSpec((NC, RB, D), lambda i: (0, i, 0))
_w_spec = pl.BlockSpec((D, D), lambda i: (0, 0))
_b_spec = pl.BlockSpec((1, D), lambda i: (0, 0))

_tc1 = pl.pallas_call(
    _tc1_body,
    grid=(N // RB,),
    in_specs=[_row_spec, _w_spec, _degp_spec],
    out_specs=_row_spec,
    out_shape=jax.ShapeDtypeStruct((N, D), _F32),
)

_tc2 = pl.pallas_call(
    _tc2_body,
    grid=(N // RB,),
    in_specs=[_aggp_spec, _row_spec, _degp_spec, _b_spec, _w_spec],
    out_specs=_row_spec,
    out_shape=jax.ShapeDtypeStruct((N, D), _F32),
)

_tc3 = pl.pallas_call(
    _tc3_body,
    grid=(N // RB,),
    in_specs=[_aggp_spec, _row_spec, _degp_spec, _b_spec,
              pl.BlockSpec((1, 1, RB), lambda i: (i, 0, 0)),
              pl.BlockSpec((D, ODIM), lambda i: (0, 0)),
              pl.BlockSpec((1, ODIM), lambda i: (0, 0))],
    out_specs=pl.BlockSpec((G, ODIM), lambda i: (0, 0)),
    out_shape=jax.ShapeDtypeStruct((G, ODIM), _F32),
    scratch_shapes=[pltpu.VMEM((G, D), _F32), pltpu.VMEM((G, D), _F32)],
)


def kernel(x, edge_index, batch_index, W1, b1, W2, b2, Wlin, blin):
    e0 = edge_index[0].astype(jnp.int32).reshape(NW, EPW)
    e1 = edge_index[1].astype(jnp.int32).reshape(NW, EPW)
    padk = jnp.arange(NW * PADE, dtype=jnp.int32).reshape(NW, PADE)
    src = jnp.concatenate([e0, padk % N], axis=1).reshape(NW, NCHUNK, CHUNK)
    dst = jnp.concatenate([e1, N + padk % PADE], axis=1).reshape(NW, NCHUNK, CHUNK)
    sdi = jnp.stack([src, dst], axis=2)
    bi3d = batch_index.astype(jnp.int32).reshape(N // RB, 1, RB)
    ones_chunk = jnp.ones((CHUNK,), _F32)
    zeros_agg = jnp.zeros((N, D), _F32)

    deg_sc, agg_sc = _build_sc_kernels()
    deg1d = deg_sc(dst, ones_chunk)
    degp = deg1d.reshape(NC, NP)[:, :N].transpose(1, 0)
    hs1 = _tc1(x, W1, degp)
    aggp1 = agg_sc(hs1, sdi, zeros_agg)
    hs2 = _tc2(aggp1, hs1, degp, b1.reshape(1, D), W2)
    aggp2 = agg_sc(hs2, sdi, zeros_agg)
    logits = _tc3(aggp2, hs2, degp, b2.reshape(1, D), bi3d,
                  Wlin, blin.reshape(1, ODIM))
    return logits
